# speedup vs baseline: 2.2346x; 2.2346x over previous
import functools

import jax
import jax.numpy as jnp
from jax import lax
from jax.experimental import pallas as pl
from jax.experimental.pallas import tpu as pltpu
from jax.experimental.pallas import tpu_sc as plsc

_BATCH = 16384
_COND = 128


def kernel(labels, embedding_table):
    info = plsc.get_sparse_core_info()
    nw = info.num_cores * info.num_subcores
    b_per_w = _BATCH // nw

    mesh = plsc.VectorSubcoreMesh(core_axis_name="c", subcore_axis_name="s")

    @functools.partial(
        pl.kernel,
        mesh=mesh,
        out_type=jax.ShapeDtypeStruct((_BATCH, _COND), jnp.float32),
        scratch_types=[
            pltpu.VMEM((b_per_w,), jnp.int32),
            pltpu.VMEM((b_per_w, _COND), jnp.float32),
            pltpu.SemaphoreType.DMA,
        ],
    )
    def emb(labels_hbm, table_hbm, out_hbm, idx_v, rows_v, sem):
        wid = lax.axis_index("s") * info.num_cores + lax.axis_index("c")
        base = wid * b_per_w
        pltpu.sync_copy(labels_hbm.at[pl.ds(base, b_per_w)], idx_v)
        pltpu.async_copy(table_hbm.at[idx_v], rows_v, sem).wait()
        pltpu.sync_copy(rows_v, out_hbm.at[pl.ds(base, b_per_w)])

    return emb(labels.astype(jnp.int32), embedding_table)


# speedup vs baseline: 2.2424x; 1.0035x over previous
import functools

import jax
import jax.numpy as jnp
from jax import lax
from jax.experimental import pallas as pl
from jax.experimental.pallas import tpu as pltpu
from jax.experimental.pallas import tpu_sc as plsc

_BATCH = 16384
_COND = 128


def kernel(labels, embedding_table):
    info = plsc.get_sparse_core_info()
    nw = info.num_cores * info.num_subcores
    b_per_w = _BATCH // nw

    mesh = plsc.VectorSubcoreMesh(core_axis_name="c", subcore_axis_name="s")

    n_chunks = 4
    chunk = b_per_w // n_chunks

    @functools.partial(
        pl.kernel,
        mesh=mesh,
        out_type=jax.ShapeDtypeStruct((_BATCH, _COND), jnp.float32),
        scratch_types=[
            pltpu.VMEM((b_per_w,), jnp.int32),
            pltpu.VMEM((n_chunks, chunk, _COND), jnp.float32),
            pltpu.SemaphoreType.DMA((n_chunks,)),
            pltpu.SemaphoreType.DMA,
        ],
    )
    def emb(labels_hbm, table_hbm, out_hbm, idx_v, bufs, gsems, wsem):
        wid = lax.axis_index("s") * info.num_cores + lax.axis_index("c")
        base = wid * b_per_w
        pltpu.sync_copy(labels_hbm.at[pl.ds(base, b_per_w)], idx_v)
        gathers = [
            pltpu.async_copy(
                table_hbm.at[idx_v.at[pl.ds(c * chunk, chunk)]],
                bufs.at[c],
                gsems.at[c],
            )
            for c in range(n_chunks)
        ]
        writes = []
        for c in range(n_chunks):
            gathers[c].wait()
            writes.append(
                pltpu.async_copy(
                    bufs.at[c], out_hbm.at[pl.ds(base + c * chunk, chunk)], wsem
                )
            )
        for w in writes:
            w.wait()

    return emb(labels.astype(jnp.int32), embedding_table)


# speedup vs baseline: 2.7205x; 1.2132x over previous
import functools

import jax
import jax.numpy as jnp
from jax import lax
from jax.experimental import pallas as pl
from jax.experimental.pallas import tpu as pltpu
from jax.experimental.pallas import tpu_sc as plsc

_BATCH = 16384
_COND = 128


def kernel(labels, embedding_table):
    info = plsc.get_sparse_core_info()
    nw = info.num_cores * info.num_subcores
    b_per_w = _BATCH // nw

    mesh = plsc.VectorSubcoreMesh(core_axis_name="c", subcore_axis_name="s")

    n_chunks = 4
    chunk = b_per_w // n_chunks

    @functools.partial(
        pl.kernel,
        mesh=mesh,
        out_type=jax.ShapeDtypeStruct((_BATCH, _COND), jnp.float32),
        scratch_types=[
            pltpu.VMEM((b_per_w,), jnp.int32),
            pltpu.VMEM((n_chunks, chunk, _COND), jnp.float32),
            pltpu.VMEM_SHARED((1001, _COND), jnp.float32),
            pltpu.SemaphoreType.DMA((n_chunks,)),
            pltpu.SemaphoreType.DMA,
        ],
    )
    def emb(labels_hbm, table_hbm, out_hbm, idx_v, bufs, table_sp, gsems, wsem):
        wid = lax.axis_index("s") * info.num_cores + lax.axis_index("c")
        base = wid * b_per_w
        sid = lax.axis_index("s")

        @pl.when(sid == 0)
        def _stage_table():
            pltpu.sync_copy(table_hbm, table_sp)

        pltpu.sync_copy(labels_hbm.at[pl.ds(base, b_per_w)], idx_v)
        plsc.subcore_barrier()
        gathers = [
            pltpu.async_copy(
                table_sp.at[idx_v.at[pl.ds(c * chunk, chunk)]],
                bufs.at[c],
                gsems.at[c],
            )
            for c in range(n_chunks)
        ]
        writes = []
        for c in range(n_chunks):
            gathers[c].wait()
            writes.append(
                pltpu.async_copy(
                    bufs.at[c], out_hbm.at[pl.ds(base + c * chunk, chunk)], wsem
                )
            )
        for w in writes:
            w.wait()

    return emb(labels.astype(jnp.int32), embedding_table)


# speedup vs baseline: 2.7328x; 1.0045x over previous
import functools

import jax
import jax.numpy as jnp
from jax import lax
from jax.experimental import pallas as pl
from jax.experimental.pallas import tpu as pltpu
from jax.experimental.pallas import tpu_sc as plsc

_BATCH = 16384
_COND = 128


def kernel(labels, embedding_table):
    info = plsc.get_sparse_core_info()
    nw = info.num_cores * info.num_subcores
    b_per_w = _BATCH // nw

    mesh = plsc.VectorSubcoreMesh(core_axis_name="c", subcore_axis_name="s")

    n_chunks = 8
    chunk = b_per_w // n_chunks

    @functools.partial(
        pl.kernel,
        mesh=mesh,
        out_type=jax.ShapeDtypeStruct((_BATCH, _COND), jnp.float32),
        scratch_types=[
            pltpu.VMEM((b_per_w,), jnp.int32),
            pltpu.VMEM((n_chunks, chunk, _COND), jnp.float32),
            pltpu.VMEM_SHARED((1001, _COND), jnp.float32),
            pltpu.SemaphoreType.DMA((n_chunks,)),
            pltpu.SemaphoreType.DMA,
        ],
    )
    def emb(labels_hbm, table_hbm, out_hbm, idx_v, bufs, table_sp, gsems, wsem):
        wid = lax.axis_index("s") * info.num_cores + lax.axis_index("c")
        base = wid * b_per_w
        sid = lax.axis_index("s")

        @pl.when(sid == 0)
        def _stage_table():
            pltpu.sync_copy(table_hbm, table_sp)

        pltpu.sync_copy(labels_hbm.at[pl.ds(base, b_per_w)], idx_v)
        plsc.subcore_barrier()
        gathers = [
            pltpu.async_copy(
                table_sp.at[idx_v.at[pl.ds(c * chunk, chunk)]],
                bufs.at[c],
                gsems.at[c],
            )
            for c in range(n_chunks)
        ]
        writes = []
        for c in range(n_chunks):
            gathers[c].wait()
            writes.append(
                pltpu.async_copy(
                    bufs.at[c], out_hbm.at[pl.ds(base + c * chunk, chunk)], wsem
                )
            )
        for w in writes:
            w.wait()

    return emb(labels.astype(jnp.int32), embedding_table)


# speedup vs baseline: 2.7506x; 1.0065x over previous
import functools

import jax
import jax.numpy as jnp
from jax import lax
from jax.experimental import pallas as pl
from jax.experimental.pallas import tpu as pltpu
from jax.experimental.pallas import tpu_sc as plsc

_BATCH = 16384
_COND = 128


def kernel(labels, embedding_table):
    info = plsc.get_sparse_core_info()
    nw = info.num_cores * info.num_subcores
    b_per_w = _BATCH // nw

    mesh = plsc.VectorSubcoreMesh(core_axis_name="c", subcore_axis_name="s")

    n_chunks = 8
    chunk = b_per_w // n_chunks

    @functools.partial(
        pl.kernel,
        mesh=mesh,
        out_type=jax.ShapeDtypeStruct((_BATCH, _COND), jnp.float32),
        scratch_types=[
            pltpu.VMEM((b_per_w,), jnp.int32),
            pltpu.VMEM((n_chunks, chunk, _COND), jnp.float32),
            pltpu.VMEM_SHARED((1001, _COND), jnp.float32),
            pltpu.SemaphoreType.DMA((n_chunks,)),
            pltpu.SemaphoreType.DMA,
        ],
    )
    def emb(labels_hbm, table_hbm, out_hbm, idx_v, bufs, table_sp, gsems, wsem):
        wid = lax.axis_index("s") * info.num_cores + lax.axis_index("c")
        base = wid * b_per_w
        sid = lax.axis_index("s")

        @pl.when(sid < 15)
        def _stage_table():
            pltpu.sync_copy(
                table_hbm.at[pl.ds(sid * 64, 64)], table_sp.at[pl.ds(sid * 64, 64)]
            )

        @pl.when(sid == 15)
        def _stage_table_tail():
            pltpu.sync_copy(table_hbm.at[pl.ds(960, 41)], table_sp.at[pl.ds(960, 41)])

        pltpu.sync_copy(labels_hbm.at[pl.ds(base, b_per_w)], idx_v)
        plsc.subcore_barrier()
        gathers = [
            pltpu.async_copy(
                table_sp.at[idx_v.at[pl.ds(c * chunk, chunk)]],
                bufs.at[c],
                gsems.at[c],
            )
            for c in range(n_chunks)
        ]
        writes = []
        for c in range(n_chunks):
            gathers[c].wait()
            writes.append(
                pltpu.async_copy(
                    bufs.at[c], out_hbm.at[pl.ds(base + c * chunk, chunk)], wsem
                )
            )
        for w in writes:
            w.wait()

    return emb(labels.astype(jnp.int32), embedding_table)


# speedup vs baseline: 2.8194x; 1.0250x over previous
import functools

import jax
import jax.numpy as jnp
from jax import lax
from jax.experimental import pallas as pl
from jax.experimental.pallas import tpu as pltpu
from jax.experimental.pallas import tpu_sc as plsc

_BATCH = 16384
_COND = 128


def kernel(labels, embedding_table):
    info = plsc.get_sparse_core_info()
    nw = info.num_cores * info.num_subcores
    b_per_w = _BATCH // nw

    mesh = plsc.VectorSubcoreMesh(core_axis_name="c", subcore_axis_name="s")

    n_chunks = 8
    chunk = b_per_w // n_chunks

    @functools.partial(
        pl.kernel,
        mesh=mesh,
        out_type=jax.ShapeDtypeStruct((_BATCH, _COND), jnp.float32),
        scratch_types=[
            pltpu.VMEM((b_per_w,), jnp.int32),
            pltpu.VMEM((n_chunks, chunk, _COND), jnp.float32),
            pltpu.VMEM_SHARED((1001, _COND), jnp.float32),
            pltpu.SemaphoreType.DMA((n_chunks,)),
            pltpu.SemaphoreType.DMA,
            pltpu.SemaphoreType.DMA,
        ],
    )
    def emb(labels_hbm, table_hbm, out_hbm, idx_v, bufs, table_sp, gsems, wsem, isem):
        wid = lax.axis_index("s") * info.num_cores + lax.axis_index("c")
        base = wid * b_per_w
        sid = lax.axis_index("s")

        idx_cp = pltpu.async_copy(labels_hbm.at[pl.ds(base, b_per_w)], idx_v, isem)

        @pl.when(sid < 15)
        def _stage_table():
            pltpu.sync_copy(
                table_hbm.at[pl.ds(sid * 64, 64)], table_sp.at[pl.ds(sid * 64, 64)]
            )

        @pl.when(sid == 15)
        def _stage_table_tail():
            pltpu.sync_copy(table_hbm.at[pl.ds(960, 41)], table_sp.at[pl.ds(960, 41)])

        idx_cp.wait()
        plsc.subcore_barrier()
        gathers = [
            pltpu.async_copy(
                table_sp.at[idx_v.at[pl.ds(c * chunk, chunk)]],
                bufs.at[c],
                gsems.at[c],
            )
            for c in range(n_chunks)
        ]
        writes = []
        for c in range(n_chunks):
            gathers[c].wait()
            writes.append(
                pltpu.async_copy(
                    bufs.at[c], out_hbm.at[pl.ds(base + c * chunk, chunk)], wsem
                )
            )
        for w in writes:
            w.wait()

    return emb(labels.astype(jnp.int32), embedding_table)
